# Initial kernel scaffold; baseline (speedup 1.0000x reference)
import jax
import jax.numpy as jnp
from jax.experimental import pallas as pl


def kernel(q, p, dq, dp, m, t, dt, length, k, Wqe1, bqe1, Wqe2, bqe2, Wqn1, bqn1, Wqn2, bqn2, Wpe1, bpe1, Wpe2, bpe2, Wpn1, bpn1, Wpn2, bpn2):
    raise NotImplementedError("write your pallas kernel here")


# speedup vs baseline: 875.4916x; 875.4916x over previous
import jax
import jax.numpy as jnp
from jax import lax
from jax.experimental import pallas as pl
from jax.experimental.pallas import tpu as pltpu


def _dot(a, b, ca, cb):
    return lax.dot_general(a, b, ((( ca,), (cb,)), ((), ())),
                           precision=lax.Precision.HIGHEST,
                           preferred_element_type=jnp.float32)


def _branch(LT, KT, xT, We1T, be1, We2T, be2, Wn1T, bn1, Wn2T, bn2, n):
    cs2 = _dot(xT, We1T[:, 0:3], 0, 1)
    baseT = _dot(We1T[:, 3:6], xT, 1, 0) + be1
    wlp = jnp.broadcast_to(We1T[:, 6:7], baseT.shape)
    wkp = jnp.broadcast_to(We1T[:, 7:8], baseT.shape)
    a3 = (LT[:, None, :] * wlp[None] + KT[:, None, :] * wkp[None]
          + cs2[:, :, None] + baseT[None])
    agg0T = jnp.maximum(a3, 0.0).sum(axis=0)
    aggT = _dot(We2T, agg0T, 1, 0) + n * be2
    gT = jnp.maximum(_dot(Wn1T[:, 0:3], xT, 1, 0)
                     + _dot(Wn1T[:, 3:], aggT, 1, 0) + bn1, 0.0)
    return _dot(Wn2T, gT, 1, 0) + bn2


def _body(LT_ref, KT_ref, qT_ref, dqT_ref, pT_ref, dpT_ref, mT_ref,
          We1Tq_ref, be1q_ref, We2Tq_ref, be2q_ref,
          Wn1Tq_ref, bn1q_ref, Wn2Tq_ref, bn2q_ref,
          We1Tp_ref, be1p_ref, We2Tp_ref, be2p_ref,
          Wn1Tp_ref, bn1p_ref, Wn2Tp_ref, bn2p_ref,
          hqT_ref, hpT_ref):
    LT = LT_ref[0]
    KT = KT_ref[0]
    mrow = mT_ref[0]
    n = LT.shape[0]
    s_count = qT_ref.shape[1]
    for s in range(s_count):
        xTq = jnp.concatenate([qT_ref[0, s:s + 1, :], dqT_ref[0, s:s + 1, :],
                               mrow], axis=0)
        xTp = jnp.concatenate([pT_ref[0, s:s + 1, :], dpT_ref[0, s:s + 1, :],
                               mrow], axis=0)
        outq = _branch(LT, KT, xTq, We1Tq_ref[...], be1q_ref[...],
                       We2Tq_ref[...], be2q_ref[...], Wn1Tq_ref[...],
                       bn1q_ref[...], Wn2Tq_ref[...], bn2q_ref[...], n)
        outp = _branch(LT, KT, xTp, We1Tp_ref[...], be1p_ref[...],
                       We2Tp_ref[...], be2p_ref[...], Wn1Tp_ref[...],
                       bn1p_ref[...], Wn2Tp_ref[...], bn2p_ref[...], n)
        hqT_ref[0, s, :] = outq[0]
        hpT_ref[0, s, :] = outp[0]


def kernel(q, p, dq, dp, m, t, dt, length, k,
           Wqe1, bqe1, Wqe2, bqe2, Wqn1, bqn1, Wqn2, bqn2,
           Wpe1, bpe1, Wpe2, bpe2, Wpn1, bpn1, Wpn2, bpn2):
    B, N, S = q.shape
    H = Wqe1.shape[1]
    f32 = jnp.float32

    LT = jnp.swapaxes(length, 1, 2)
    KT = jnp.swapaxes(k, 1, 2)
    qT = jnp.swapaxes(q, 1, 2)
    dqT = jnp.swapaxes(dq, 1, 2)
    pT = jnp.swapaxes(p, 1, 2)
    dpT = jnp.swapaxes(dp, 1, 2)
    mT = jnp.swapaxes(m, 1, 2)

    wargs = (Wqe1.T, bqe1[:, None], Wqe2.T, bqe2[:, None],
             Wqn1.T, bqn1[:, None], Wqn2.T, bqn2[:, None],
             Wpe1.T, bpe1[:, None], Wpe2.T, bpe2[:, None],
             Wpn1.T, bpn1[:, None], Wpn2.T, bpn2[:, None])

    def bspec(shape3):
        return pl.BlockSpec(shape3, lambda b: (b, 0, 0))

    def wspec(arr):
        sh = arr.shape
        return pl.BlockSpec(sh, lambda b: tuple(0 for _ in sh))

    grid_spec = pl.GridSpec(
        grid=(B,),
        in_specs=[bspec((1, N, N)), bspec((1, N, N)),
                  bspec((1, S, N)), bspec((1, S, N)),
                  bspec((1, S, N)), bspec((1, S, N)),
                  bspec((1, 1, N))] + [wspec(w) for w in wargs],
        out_specs=[bspec((1, S, N)), bspec((1, S, N))],
    )

    hqT, hpT = pl.pallas_call(
        _body,
        grid_spec=grid_spec,
        out_shape=[jax.ShapeDtypeStruct((B, S, N), f32),
                   jax.ShapeDtypeStruct((B, S, N), f32)],
        compiler_params=pltpu.CompilerParams(
            dimension_semantics=("arbitrary",)),
    )(LT, KT, qT, dqT, pT, dpT, mT, *wargs)

    return jnp.swapaxes(hqT, 1, 2), jnp.swapaxes(hpT, 1, 2)
